# baseline (speedup 1.0000x reference)
import jax
import jax.numpy as jnp
from jax.experimental import pallas as pl
from jax.experimental.pallas import tpu as pltpu

_N_CTX = 12
_SUF = 64
_L = 77
_D = 768
_TOT = 2 * _L * _D

_OFF = (0, _D, (1 + _N_CTX) * _D,
        _L * _D, (_L + 1) * _D, (_L + 1 + _N_CTX) * _D)


def _assemble_body(pp, cp, sp, pn, cn, sn, tp, tn, out_p, out_t, *sems):
    srcs = (pp, cp, sp, pn, cn, sn)
    copies = [
        pltpu.make_async_copy(src, out_p.at[0:1, pl.ds(off, src.shape[1])], sem)
        for src, off, sem in zip(srcs, _OFF, sems)
    ]
    for c in copies:
        c.start()
    out_t[0:1, :] = tp[...]
    out_t[1:2, :] = tn[...]
    for c in copies:
        c.wait()


def kernel(ctx_pos, ctx_neg, token_prefix_pos, token_suffix_pos,
           token_prefix_neg, token_suffix_neg, tokenized_prompts_pos,
           tokenized_prompts_neg, compound_prompts_text):
    pp = token_prefix_pos.reshape(1, _D)
    cp = ctx_pos.reshape(1, _N_CTX * _D)
    sp = token_suffix_pos.reshape(1, _SUF * _D)
    pn = token_prefix_neg.reshape(1, _D)
    cn = ctx_neg.reshape(1, _N_CTX * _D)
    sn = token_suffix_neg.reshape(1, _SUF * _D)
    tp = tokenized_prompts_pos.reshape(1, _L)
    tn = tokenized_prompts_neg.reshape(1, _L)

    vmem = pl.BlockSpec(memory_space=pltpu.MemorySpace.VMEM)
    any_spec = pl.BlockSpec(memory_space=pl.ANY)
    prompts_row, tok = pl.pallas_call(
        _assemble_body,
        in_specs=[vmem] * 8,
        out_specs=(any_spec, vmem),
        out_shape=(
            jax.ShapeDtypeStruct((1, _TOT), jnp.float32),
            jax.ShapeDtypeStruct((2, _L), jnp.int32),
        ),
        scratch_shapes=[pltpu.SemaphoreType.DMA] * 6,
    )(pp, cp, sp, pn, cn, sn, tp, tn)

    return prompts_row.reshape(2, _L, _D), tok, compound_prompts_text


# speedup vs baseline: 1.0627x; 1.0627x over previous
import jax
import jax.numpy as jnp
from jax.experimental import pallas as pl
from jax.experimental.pallas import tpu as pltpu

_N_CTX = 12
_SUF = 64
_L = 77
_D = 768
_TOT = 2 * _L * _D

_OFF = (0, _D, (1 + _N_CTX) * _D,
        _L * _D, (_L + 1) * _D, (_L + 1 + _N_CTX) * _D)


def _assemble_body(pp, cp, sp, pn, cn, sn, tp, tn, out_p, out_t):
    for src, off in zip((pp, cp, sp, pn, cn, sn), _OFF):
        out_p[0:1, pl.ds(off, src.shape[1])] = src[...]
    out_t[0:1, :] = tp[...]
    out_t[1:2, :] = tn[...]


def kernel(ctx_pos, ctx_neg, token_prefix_pos, token_suffix_pos,
           token_prefix_neg, token_suffix_neg, tokenized_prompts_pos,
           tokenized_prompts_neg, compound_prompts_text):
    pp = token_prefix_pos.reshape(1, _D)
    cp = ctx_pos.reshape(1, _N_CTX * _D)
    sp = token_suffix_pos.reshape(1, _SUF * _D)
    pn = token_prefix_neg.reshape(1, _D)
    cn = ctx_neg.reshape(1, _N_CTX * _D)
    sn = token_suffix_neg.reshape(1, _SUF * _D)
    tp = tokenized_prompts_pos.reshape(1, _L)
    tn = tokenized_prompts_neg.reshape(1, _L)

    prompts_row, tok = pl.pallas_call(
        _assemble_body,
        out_shape=(
            jax.ShapeDtypeStruct((1, _TOT), jnp.float32),
            jax.ShapeDtypeStruct((2, _L), jnp.int32),
        ),
    )(pp, cp, sp, pn, cn, sn, tp, tn)

    return prompts_row.reshape(2, _L, _D), tok, compound_prompts_text
